# baseline (speedup 1.0000x reference)
import functools

import jax
import jax.numpy as jnp
from jax import lax
from jax.experimental import pallas as pl
from jax.experimental.pallas import tpu as pltpu
from jax.experimental.pallas import tpu_sc as plsc

D = H = W = 256
N_RAYS, N_BINS = 16384, 128
N = N_RAYS * N_BINS
NC, NS, L = 2, 16, 16
NW = NC * NS
P = N // NW
C = 4096
NCHUNK = P // C


def _floor_i32(x):
    t = x.astype(jnp.int32)
    tf = t.astype(jnp.float32)
    return jnp.where(tf > x, t - 1, t)


def _axis(c, extent):
    x = c * (0.5 * extent) + (0.5 * extent - 0.5)
    i0 = _floor_i32(x)
    i1 = i0 + 1
    w1 = x - i0.astype(jnp.float32)
    w0 = 1.0 - w1
    hi = extent - 1
    v0 = (i0 >= 0) & (i0 <= hi)
    v1 = (i1 >= 0) & (i1 <= hi)
    w0 = jnp.where(v0, w0, 0.0)
    w1 = jnp.where(v1, w1, 0.0)
    c0 = jnp.minimum(jnp.maximum(i0, 0), hi)
    c1 = jnp.minimum(jnp.maximum(i1, 0), hi)
    return c0, c1, w0, w1


def _body(grid_hbm, coords_hbm, out_hbm, coords_v, idx_v, w_v, vals_v,
          out_v, sem):
    wid = lax.axis_index("s") * NC + lax.axis_index("c")
    base_pt = wid * P
    lanes = lax.iota(jnp.int32, L)

    def chunk_body(ci, carry):
        pt0 = base_pt + ci * C
        pltpu.sync_copy(coords_hbm.at[pl.ds(pt0 * 3, 3 * C)], coords_v)

        def grp(g, carry2):
            o = g * L
            p3 = (o + lanes) * 3
            cx = plsc.load_gather(coords_v, [p3])
            cy = plsc.load_gather(coords_v, [p3 + 1])
            cz = plsc.load_gather(coords_v, [p3 + 2])
            x0, x1, wx0, wx1 = _axis(cx, W)
            y0, y1, wy0, wy1 = _axis(cy, H)
            z0, z1, wz0, wz1 = _axis(cz, D)
            iz0 = z0 * (H * 128)
            iz1 = z1 * (H * 128)
            iy0 = y0 * 128
            iy1 = y1 * 128
            x0 = (x0 & 127) + (x0 >> 7) * (D * H * 128)
            x1 = (x1 & 127) + (x1 >> 7) * (D * H * 128)
            w00 = wz0 * wy0
            w01 = wz0 * wy1
            w10 = wz1 * wy0
            w11 = wz1 * wy1
            corners = (
                (iz0 + iy0 + x0, w00 * wx0),
                (iz0 + iy0 + x1, w00 * wx1),
                (iz0 + iy1 + x0, w01 * wx0),
                (iz0 + iy1 + x1, w01 * wx1),
                (iz1 + iy0 + x0, w10 * wx0),
                (iz1 + iy0 + x1, w10 * wx1),
                (iz1 + iy1 + x0, w11 * wx0),
                (iz1 + iy1 + x1, w11 * wx1),
            )
            for k, (iv, wv) in enumerate(corners):
                idx_v[pl.ds(k * C + o, L)] = iv
                w_v[pl.ds(k * C + o, L)] = wv
            return carry2

        lax.fori_loop(0, C // L, grp, 0, unroll=2)

        pltpu.async_copy(grid_hbm.at[idx_v], vals_v, sem).wait()

        def grp2(g, carry2):
            o = g * L
            acc = vals_v[pl.ds(o, L)] * w_v[pl.ds(o, L)]
            for k in range(1, 8):
                acc = acc + vals_v[pl.ds(k * C + o, L)] * w_v[pl.ds(k * C + o, L)]
            out_v[pl.ds(o, L)] = acc
            return carry2

        lax.fori_loop(0, C // L, grp2, 0, unroll=2)
        pltpu.sync_copy(out_v, out_hbm.at[pl.ds(pt0, C)])
        return carry

    lax.fori_loop(0, NCHUNK, chunk_body, 0)


R_BLK = 2048
N_RB = D * H // R_BLK


def _detile_body(x_ref, o_ref):
    o_ref[...] = x_ref[...]


def _detile_grid(grid2):
    out = pl.pallas_call(
        _detile_body,
        grid=(N_RB, 2),
        in_specs=[pl.BlockSpec((R_BLK, 128), lambda i, j: (i, j))],
        out_specs=pl.BlockSpec((R_BLK, 128), lambda i, j: (j * N_RB + i, 0)),
        out_shape=jax.ShapeDtypeStruct((2 * D * H, 128), jnp.float32),
    )(grid2)
    return out.reshape(D * H * W)


@jax.jit
def kernel(occupancy_grid, ray_bin_centers):
    grid_flat = _detile_grid(occupancy_grid.reshape(D * H, W))
    coords_flat = ray_bin_centers.reshape(N * 3)
    mesh = plsc.VectorSubcoreMesh(core_axis_name="c", subcore_axis_name="s")
    run = pl.kernel(
        _body,
        out_type=jax.ShapeDtypeStruct((N,), jnp.float32),
        mesh=mesh,
        compiler_params=pltpu.CompilerParams(needs_layout_passes=False),
        scratch_types=[
            pltpu.VMEM((3 * C,), jnp.float32),
            pltpu.VMEM((8 * C,), jnp.int32),
            pltpu.VMEM((8 * C,), jnp.float32),
            pltpu.VMEM((8 * C,), jnp.float32),
            pltpu.VMEM((C,), jnp.float32),
            pltpu.SemaphoreType.DMA,
        ],
    )
    out = run(grid_flat, coords_flat)
    return out.reshape(N_RAYS, N_BINS)


# speedup vs baseline: 4.2397x; 4.2397x over previous
import functools

import jax
import jax.numpy as jnp
from jax import lax
from jax.experimental import pallas as pl
from jax.experimental.pallas import tpu as pltpu
from jax.experimental.pallas import tpu_sc as plsc

D = H = W = 256
N_RAYS, N_BINS = 16384, 128
N = N_RAYS * N_BINS
NC, NS, L = 2, 16, 16
NW = NC * NS
P = N // NW
C = 4096
NCHUNK = P // C


def _floor_i32(x):
    t = x.astype(jnp.int32)
    tf = t.astype(jnp.float32)
    return jnp.where(tf > x, t - 1, t)


def _axis(c, extent):
    x = c * (0.5 * extent) + (0.5 * extent - 0.5)
    i0 = _floor_i32(x)
    i1 = i0 + 1
    w1 = x - i0.astype(jnp.float32)
    w0 = 1.0 - w1
    hi = extent - 1
    v0 = (i0 >= 0) & (i0 <= hi)
    v1 = (i1 >= 0) & (i1 <= hi)
    w0 = jnp.where(v0, w0, 0.0)
    w1 = jnp.where(v1, w1, 0.0)
    c0 = jnp.minimum(jnp.maximum(i0, 0), hi)
    c1 = jnp.minimum(jnp.maximum(i1, 0), hi)
    return c0, c1, w0, w1


def _body(grid_hbm, coords_hbm, out_hbm, coords_v, idx_v, w_v, vals_v,
          out_v, sem):
    wid = lax.axis_index("s") * NC + lax.axis_index("c")
    base_pt = wid * P
    lanes = lax.iota(jnp.int32, L)

    def chunk_body(ci, carry):
        pt0 = base_pt + ci * C
        pltpu.sync_copy(coords_hbm.at[pl.ds(pt0, C)], coords_v.at[pl.ds(0, C)])
        pltpu.sync_copy(coords_hbm.at[pl.ds(N + pt0, C)],
                        coords_v.at[pl.ds(C, C)])
        pltpu.sync_copy(coords_hbm.at[pl.ds(2 * N + pt0, C)],
                        coords_v.at[pl.ds(2 * C, C)])

        def grp(g, carry2):
            o = g * L
            cx = coords_v[pl.ds(o, L)]
            cy = coords_v[pl.ds(C + o, L)]
            cz = coords_v[pl.ds(2 * C + o, L)]
            x0, x1, wx0, wx1 = _axis(cx, W)
            y0, y1, wy0, wy1 = _axis(cy, H)
            z0, z1, wz0, wz1 = _axis(cz, D)
            iz0 = z0 * (H * 128)
            iz1 = z1 * (H * 128)
            iy0 = y0 * 128
            iy1 = y1 * 128
            x0 = (x0 & 127) + (x0 >> 7) * (D * H * 128)
            x1 = (x1 & 127) + (x1 >> 7) * (D * H * 128)
            w00 = wz0 * wy0
            w01 = wz0 * wy1
            w10 = wz1 * wy0
            w11 = wz1 * wy1
            corners = (
                (iz0 + iy0 + x0, w00 * wx0),
                (iz0 + iy0 + x1, w00 * wx1),
                (iz0 + iy1 + x0, w01 * wx0),
                (iz0 + iy1 + x1, w01 * wx1),
                (iz1 + iy0 + x0, w10 * wx0),
                (iz1 + iy0 + x1, w10 * wx1),
                (iz1 + iy1 + x0, w11 * wx0),
                (iz1 + iy1 + x1, w11 * wx1),
            )
            for k, (iv, wv) in enumerate(corners):
                idx_v[pl.ds(k * C + o, L)] = iv
                w_v[pl.ds(k * C + o, L)] = wv
            return carry2

        lax.fori_loop(0, C // L, grp, 0, unroll=2)

        pltpu.async_copy(grid_hbm.at[idx_v], vals_v, sem).wait()

        def grp2(g, carry2):
            o = g * L
            acc = vals_v[pl.ds(o, L)] * w_v[pl.ds(o, L)]
            for k in range(1, 8):
                acc = acc + vals_v[pl.ds(k * C + o, L)] * w_v[pl.ds(k * C + o, L)]
            out_v[pl.ds(o, L)] = acc
            return carry2

        lax.fori_loop(0, C // L, grp2, 0, unroll=2)
        pltpu.sync_copy(out_v, out_hbm.at[pl.ds(pt0, C)])
        return carry

    lax.fori_loop(0, NCHUNK, chunk_body, 0)


R_BLK = 2048
N_RB = D * H // R_BLK


def _detile_body(x_ref, o_ref):
    o_ref[...] = x_ref[...]


def _detile_grid(grid2):
    out = pl.pallas_call(
        _detile_body,
        grid=(N_RB, 2),
        in_specs=[pl.BlockSpec((R_BLK, 128), lambda i, j: (i, j))],
        out_specs=pl.BlockSpec((R_BLK, 128), lambda i, j: (j * N_RB + i, 0)),
        out_shape=jax.ShapeDtypeStruct((2 * D * H, 128), jnp.float32),
    )(grid2)
    return out.reshape(D * H * W)


@jax.jit
def kernel(occupancy_grid, ray_bin_centers):
    grid_flat = _detile_grid(occupancy_grid.reshape(D * H, W))
    coords_flat = jnp.moveaxis(ray_bin_centers, -1, 0).reshape(3 * N)
    mesh = plsc.VectorSubcoreMesh(core_axis_name="c", subcore_axis_name="s")
    run = pl.kernel(
        _body,
        out_type=jax.ShapeDtypeStruct((N,), jnp.float32),
        mesh=mesh,
        compiler_params=pltpu.CompilerParams(needs_layout_passes=False),
        scratch_types=[
            pltpu.VMEM((3 * C,), jnp.float32),
            pltpu.VMEM((8 * C,), jnp.int32),
            pltpu.VMEM((8 * C,), jnp.float32),
            pltpu.VMEM((8 * C,), jnp.float32),
            pltpu.VMEM((C,), jnp.float32),
            pltpu.SemaphoreType.DMA,
        ],
    )
    out = run(grid_flat, coords_flat)
    return out.reshape(N_RAYS, N_BINS)


# speedup vs baseline: 4.9822x; 1.1751x over previous
import functools

import jax
import jax.numpy as jnp
from jax import lax
from jax.experimental import pallas as pl
from jax.experimental.pallas import tpu as pltpu
from jax.experimental.pallas import tpu_sc as plsc

D = H = W = 256
N_RAYS, N_BINS = 16384, 128
N = N_RAYS * N_BINS
NC, NS, L = 2, 16, 16
NW = NC * NS
P = N // NW
C = 2048
NCHUNK = P // C


def _floor_i32(x):
    t = x.astype(jnp.int32)
    tf = t.astype(jnp.float32)
    return jnp.where(tf > x, t - 1, t)


def _axis(c, extent):
    x = c * (0.5 * extent) + (0.5 * extent - 0.5)
    i0 = _floor_i32(x)
    i1 = i0 + 1
    w1 = x - i0.astype(jnp.float32)
    w0 = 1.0 - w1
    hi = extent - 1
    v0 = (i0 >= 0) & (i0 <= hi)
    v1 = (i1 >= 0) & (i1 <= hi)
    w0 = jnp.where(v0, w0, 0.0)
    w1 = jnp.where(v1, w1, 0.0)
    c0 = jnp.minimum(jnp.maximum(i0, 0), hi)
    c1 = jnp.minimum(jnp.maximum(i1, 0), hi)
    return c0, c1, w0, w1


def _body(grid_hbm, coords_hbm, out_hbm, coords_v, idx_v, w_v, vals_v,
          out_v, sem_c0, sem_c1, sem_g0, sem_g1, sem_o0, sem_o1):
    wid = lax.axis_index("s") * NC + lax.axis_index("c")
    base_pt = wid * P
    sem_c = (sem_c0, sem_c1)
    sem_g = (sem_g0, sem_g1)
    sem_o = (sem_o0, sem_o1)

    def fire_coords(ci, s):
        pt0 = base_pt + ci * C
        for a in range(3):
            pltpu.async_copy(coords_hbm.at[pl.ds(a * N + pt0, C)],
                             coords_v.at[pl.ds((3 * s + a) * C, C)], sem_c[s])

    def wait_coords(s):
        for a in range(3):
            pltpu.make_async_copy(
                coords_hbm.at[pl.ds(0, C)],
                coords_v.at[pl.ds((3 * s + a) * C, C)], sem_c[s]).wait()

    def pass1(s):
        def grp(g, carry2):
            o = g * L
            cx = coords_v[pl.ds(3 * s * C + o, L)]
            cy = coords_v[pl.ds((3 * s + 1) * C + o, L)]
            cz = coords_v[pl.ds((3 * s + 2) * C + o, L)]
            x0, x1, wx0, wx1 = _axis(cx, W)
            y0, y1, wy0, wy1 = _axis(cy, H)
            z0, z1, wz0, wz1 = _axis(cz, D)
            iz0 = z0 * (H * 128)
            iz1 = z1 * (H * 128)
            iy0 = y0 * 128
            iy1 = y1 * 128
            x0 = (x0 & 127) + (x0 >> 7) * (D * H * 128)
            x1 = (x1 & 127) + (x1 >> 7) * (D * H * 128)
            w00 = wz0 * wy0
            w01 = wz0 * wy1
            w10 = wz1 * wy0
            w11 = wz1 * wy1
            corners = (
                (iz0 + iy0 + x0, w00 * wx0),
                (iz0 + iy0 + x1, w00 * wx1),
                (iz0 + iy1 + x0, w01 * wx0),
                (iz0 + iy1 + x1, w01 * wx1),
                (iz1 + iy0 + x0, w10 * wx0),
                (iz1 + iy0 + x1, w10 * wx1),
                (iz1 + iy1 + x0, w11 * wx0),
                (iz1 + iy1 + x1, w11 * wx1),
            )
            for k, (iv, wv) in enumerate(corners):
                idx_v[pl.ds((8 * s + k) * C + o, L)] = iv
                w_v[pl.ds((8 * s + k) * C + o, L)] = wv
            return carry2

        lax.fori_loop(0, C // L, grp, 0, unroll=2)

    def fire_gather(s):
        pltpu.async_copy(grid_hbm.at[idx_v.at[pl.ds(8 * s * C, 8 * C)]],
                         vals_v.at[pl.ds(8 * s * C, 8 * C)], sem_g[s])

    def wait_gather(s):
        pltpu.make_async_copy(grid_hbm.at[pl.ds(0, 8 * C)],
                              vals_v.at[pl.ds(8 * s * C, 8 * C)],
                              sem_g[s]).wait()

    def pass2(ci, s):
        def grp2(g, carry2):
            o = g * L
            acc = (vals_v[pl.ds(8 * s * C + o, L)]
                   * w_v[pl.ds(8 * s * C + o, L)])
            for k in range(1, 8):
                acc = acc + (vals_v[pl.ds((8 * s + k) * C + o, L)]
                             * w_v[pl.ds((8 * s + k) * C + o, L)])
            out_v[pl.ds(s * C + o, L)] = acc
            return carry2

        lax.fori_loop(0, C // L, grp2, 0, unroll=2)
        pltpu.async_copy(out_v.at[pl.ds(s * C, C)],
                         out_hbm.at[pl.ds(base_pt + ci * C, C)], sem_o[s])

    def wait_out(s):
        pltpu.make_async_copy(out_v.at[pl.ds(s * C, C)],
                              out_hbm.at[pl.ds(base_pt, C)], sem_o[s]).wait()

    fire_coords(0, 0)

    def chunk_pair(i, carry):
        for s in (0, 1):
            t = 1 - s
            ci = 2 * i + s
            wait_coords(s)
            pass1(s)
            fire_gather(s)

            @pl.when(ci < NCHUNK - 1)
            def _():
                fire_coords(ci + 1, t)

            @pl.when(ci >= 3)
            def _():
                wait_out(t)

            @pl.when(ci >= 1)
            def _():
                wait_gather(t)
                pass2(ci - 1, t)

        return carry

    lax.fori_loop(0, NCHUNK // 2, chunk_pair, 0)
    s_last = (NCHUNK - 1) & 1
    wait_gather(s_last)
    pass2(NCHUNK - 1, s_last)
    wait_out(1 - s_last)
    wait_out(s_last)


R_BLK = 2048
N_RB = D * H // R_BLK


def _detile_body(x_ref, o_ref):
    o_ref[...] = x_ref[...]


def _detile_grid(grid2):
    out = pl.pallas_call(
        _detile_body,
        grid=(N_RB, 2),
        in_specs=[pl.BlockSpec((R_BLK, 128), lambda i, j: (i, j))],
        out_specs=pl.BlockSpec((R_BLK, 128), lambda i, j: (j * N_RB + i, 0)),
        out_shape=jax.ShapeDtypeStruct((2 * D * H, 128), jnp.float32),
    )(grid2)
    return out.reshape(D * H * W)


@jax.jit
def kernel(occupancy_grid, ray_bin_centers):
    grid_flat = _detile_grid(occupancy_grid.reshape(D * H, W))
    coords_flat = jnp.moveaxis(ray_bin_centers, -1, 0).reshape(3 * N)
    mesh = plsc.VectorSubcoreMesh(core_axis_name="c", subcore_axis_name="s")
    run = pl.kernel(
        _body,
        out_type=jax.ShapeDtypeStruct((N,), jnp.float32),
        mesh=mesh,
        compiler_params=pltpu.CompilerParams(needs_layout_passes=False),
        scratch_types=[
            pltpu.VMEM((2 * 3 * C,), jnp.float32),
            pltpu.VMEM((2 * 8 * C,), jnp.int32),
            pltpu.VMEM((2 * 8 * C,), jnp.float32),
            pltpu.VMEM((2 * 8 * C,), jnp.float32),
            pltpu.VMEM((2 * C,), jnp.float32),
            pltpu.SemaphoreType.DMA,
            pltpu.SemaphoreType.DMA,
            pltpu.SemaphoreType.DMA,
            pltpu.SemaphoreType.DMA,
            pltpu.SemaphoreType.DMA,
            pltpu.SemaphoreType.DMA,
        ],
    )
    out = run(grid_flat, coords_flat)
    return out.reshape(N_RAYS, N_BINS)


# speedup vs baseline: 6.2355x; 1.2516x over previous
import functools

import jax
import jax.numpy as jnp
from jax import lax
from jax.experimental import pallas as pl
from jax.experimental.pallas import tpu as pltpu
from jax.experimental.pallas import tpu_sc as plsc

D = H = W = 256
N_RAYS, N_BINS = 16384, 128
N = N_RAYS * N_BINS
NC, NS, L = 2, 16, 16
NW = NC * NS
P = N // NW
C = 2048
NCHUNK = P // C


def _floor_i32(x):
    t = x.astype(jnp.int32)
    tf = t.astype(jnp.float32)
    return jnp.where(tf > x, t - 1, t)


def _axis(c, extent):
    x = c * (0.5 * extent) + (0.5 * extent - 0.5)
    i0 = _floor_i32(x)
    i1 = i0 + 1
    w1 = x - i0.astype(jnp.float32)
    w0 = 1.0 - w1
    hi = extent - 1
    v0 = (i0 >= 0) & (i0 <= hi)
    v1 = (i1 >= 0) & (i1 <= hi)
    w0 = jnp.where(v0, w0, 0.0)
    w1 = jnp.where(v1, w1, 0.0)
    c0 = jnp.minimum(jnp.maximum(i0, 0), hi)
    c1 = jnp.minimum(jnp.maximum(i1, 0), hi)
    return i0, c0, c1, w0, w1


def _body(grid_hbm, coords_hbm, out_hbm, coords_v, idx_v, w_v, vals_v,
          out_v, sem_c0, sem_c1, sem_g0, sem_g1, sem_o0, sem_o1):
    wid = lax.axis_index("s") * NC + lax.axis_index("c")
    base_pt = wid * P
    sem_c = (sem_c0, sem_c1)
    sem_g = (sem_g0, sem_g1)
    sem_o = (sem_o0, sem_o1)

    def fire_coords(ci, s):
        pt0 = base_pt + ci * C
        for a in range(3):
            pltpu.async_copy(coords_hbm.at[pl.ds(a * N + pt0, C)],
                             coords_v.at[pl.ds((3 * s + a) * C, C)], sem_c[s])

    def wait_coords(s):
        for a in range(3):
            pltpu.make_async_copy(
                coords_hbm.at[pl.ds(0, C)],
                coords_v.at[pl.ds((3 * s + a) * C, C)], sem_c[s]).wait()

    def pass1(s):
        def grp(g, carry2):
            o = g * L
            cx = coords_v[pl.ds(3 * s * C + o, L)]
            cy = coords_v[pl.ds((3 * s + 1) * C + o, L)]
            cz = coords_v[pl.ds((3 * s + 2) * C + o, L)]
            x0r, xa, _, wx0, wx1 = _axis(cx, W)
            _, y0, y1, wy0, wy1 = _axis(cy, H)
            _, z0, z1, wz0, wz1 = _axis(cz, D)
            wlo = jnp.where(x0r >= 0, wx0, wx1)
            whi = jnp.where(x0r >= 0, wx1, 0.0)
            xa = (xa & 127) + (xa >> 7) * HALF
            iz0 = z0 * (H * 128)
            iz1 = z1 * (H * 128)
            iy0 = y0 * 128
            iy1 = y1 * 128
            corners = (
                (iz0 + iy0 + xa, wz0 * wy0),
                (iz0 + iy1 + xa, wz0 * wy1),
                (iz1 + iy0 + xa, wz1 * wy0),
                (iz1 + iy1 + xa, wz1 * wy1),
            )
            for k, (iv, wv) in enumerate(corners):
                idx_v[pl.ds((4 * s + k) * C + o, L)] = iv
                w_v[pl.ds((6 * s + k) * C + o, L)] = wv
            w_v[pl.ds((6 * s + 4) * C + o, L)] = wlo
            w_v[pl.ds((6 * s + 5) * C + o, L)] = whi
            return carry2

        lax.fori_loop(0, C // L, grp, 0, unroll=2)

    def fire_gather(s):
        pltpu.async_copy(grid_hbm.at[idx_v.at[pl.ds(4 * s * C, 4 * C)]],
                         vals_v.at[pl.ds(4 * s * C, 4 * C)], sem_g[s])

    def wait_gather(s):
        pltpu.make_async_copy(grid_hbm.at[pl.ds(0, 4 * C)],
                              vals_v.at[pl.ds(4 * s * C, 4 * C)],
                              sem_g[s]).wait()

    def pass2(ci, s):
        himask = jnp.uint32(0xFFFF0000)

        def grp2(g, carry2):
            o = g * L
            wlo = w_v[pl.ds((6 * s + 4) * C + o, L)]
            whi = w_v[pl.ds((6 * s + 5) * C + o, L)]
            acc = None
            for k in range(4):
                u = vals_v[pl.ds((4 * s + k) * C + o, L)]
                v0 = plsc.bitcast(u << 16, jnp.float32)
                v1 = plsc.bitcast(u & himask, jnp.float32)
                wzy = w_v[pl.ds((6 * s + k) * C + o, L)]
                term = wzy * (v0 * wlo + v1 * whi)
                acc = term if acc is None else acc + term
            out_v[pl.ds(s * C + o, L)] = acc
            return carry2

        lax.fori_loop(0, C // L, grp2, 0, unroll=2)
        pltpu.async_copy(out_v.at[pl.ds(s * C, C)],
                         out_hbm.at[pl.ds(base_pt + ci * C, C)], sem_o[s])

    def wait_out(s):
        pltpu.make_async_copy(out_v.at[pl.ds(s * C, C)],
                              out_hbm.at[pl.ds(base_pt, C)], sem_o[s]).wait()

    fire_coords(0, 0)

    def chunk_pair(i, carry):
        for s in (0, 1):
            t = 1 - s
            ci = 2 * i + s
            wait_coords(s)
            pass1(s)
            fire_gather(s)

            @pl.when(ci < NCHUNK - 1)
            def _():
                fire_coords(ci + 1, t)

            @pl.when(ci >= 3)
            def _():
                wait_out(t)

            @pl.when(ci >= 1)
            def _():
                wait_gather(t)
                pass2(ci - 1, t)

        return carry

    lax.fori_loop(0, NCHUNK // 2, chunk_pair, 0)
    s_last = (NCHUNK - 1) & 1
    wait_gather(s_last)
    pass2(NCHUNK - 1, s_last)
    wait_out(1 - s_last)
    wait_out(s_last)


R_BLK = 2048
N_RB = D * H // R_BLK


def _detile_body(x_ref, o_ref):
    o_ref[...] = x_ref[...]


def _detile_grid(grid2):
    out = pl.pallas_call(
        _detile_body,
        grid=(N_RB, 2),
        in_specs=[pl.BlockSpec((R_BLK, 128), lambda i, j: (i, j))],
        out_specs=pl.BlockSpec((R_BLK, 128), lambda i, j: (j * N_RB + i, 0)),
        out_shape=jax.ShapeDtypeStruct((2 * D * H, 128), jnp.float32),
    )(grid2)
    return out.reshape(D * H * W)


HALF = D * H * 128


def _pack_pairs(g):
    s1 = jnp.roll(g, -1)
    s2 = jnp.roll(g, -(HALF - 127))
    lane = jax.lax.iota(jnp.int32, D * H * W) & 127
    nxt = jnp.where(lane == 127, s2, s1)
    lo = jax.lax.bitcast_convert_type(g.astype(jnp.bfloat16),
                                      jnp.uint16).astype(jnp.uint32)
    hi = jax.lax.bitcast_convert_type(nxt.astype(jnp.bfloat16),
                                      jnp.uint16).astype(jnp.uint32)
    return lo | (hi << 16)


@jax.jit
def kernel(occupancy_grid, ray_bin_centers):
    grid_flat = _pack_pairs(_detile_grid(occupancy_grid.reshape(D * H, W)))
    coords_flat = jnp.moveaxis(ray_bin_centers, -1, 0).reshape(3 * N)
    mesh = plsc.VectorSubcoreMesh(core_axis_name="c", subcore_axis_name="s")
    run = pl.kernel(
        _body,
        out_type=jax.ShapeDtypeStruct((N,), jnp.float32),
        mesh=mesh,
        compiler_params=pltpu.CompilerParams(needs_layout_passes=False),
        scratch_types=[
            pltpu.VMEM((2 * 3 * C,), jnp.float32),
            pltpu.VMEM((2 * 4 * C,), jnp.int32),
            pltpu.VMEM((2 * 6 * C,), jnp.float32),
            pltpu.VMEM((2 * 4 * C,), jnp.uint32),
            pltpu.VMEM((2 * C,), jnp.float32),
            pltpu.SemaphoreType.DMA,
            pltpu.SemaphoreType.DMA,
            pltpu.SemaphoreType.DMA,
            pltpu.SemaphoreType.DMA,
            pltpu.SemaphoreType.DMA,
            pltpu.SemaphoreType.DMA,
        ],
    )
    out = run(grid_flat, coords_flat)
    return out.reshape(N_RAYS, N_BINS)


# speedup vs baseline: 8.2117x; 1.3169x over previous
import functools

import jax
import jax.numpy as jnp
from jax import lax
from jax.experimental import pallas as pl
from jax.experimental.pallas import tpu as pltpu
from jax.experimental.pallas import tpu_sc as plsc

D = H = W = 256
N_RAYS, N_BINS = 16384, 128
N = N_RAYS * N_BINS
NC, NS, L = 2, 16, 16
NW = NC * NS
P = N // NW
C = 2048
NCHUNK = P // C


def _floor_i32(x):
    t = x.astype(jnp.int32)
    tf = t.astype(jnp.float32)
    return jnp.where(tf > x, t - 1, t)


def _axis(c, extent):
    x = c * (0.5 * extent) + (0.5 * extent - 0.5)
    i0 = _floor_i32(x)
    i1 = i0 + 1
    w1 = x - i0.astype(jnp.float32)
    w0 = 1.0 - w1
    hi = extent - 1
    v0 = (i0 >= 0) & (i0 <= hi)
    v1 = (i1 >= 0) & (i1 <= hi)
    w0 = jnp.where(v0, w0, 0.0)
    w1 = jnp.where(v1, w1, 0.0)
    c0 = jnp.minimum(jnp.maximum(i0, 0), hi)
    c1 = jnp.minimum(jnp.maximum(i1, 0), hi)
    return i0, c0, c1, w0, w1


def _body(grid_hbm, coords_hbm, out_hbm, coords_v, idx_v, w_v, vals_v,
          out_v, sem_c0, sem_c1, sem_g0, sem_g1, sem_o0, sem_o1):
    wid = lax.axis_index("s") * NC + lax.axis_index("c")
    base_pt = wid * P
    sem_c = (sem_c0, sem_c1)
    sem_g = (sem_g0, sem_g1)
    sem_o = (sem_o0, sem_o1)

    def fire_coords(ci, s):
        pt0 = base_pt + ci * C
        for a in range(3):
            pltpu.async_copy(coords_hbm.at[pl.ds(a * N + pt0, C)],
                             coords_v.at[pl.ds((3 * s + a) * C, C)], sem_c[s])

    def wait_coords(s):
        for a in range(3):
            pltpu.make_async_copy(
                coords_hbm.at[pl.ds(0, C)],
                coords_v.at[pl.ds((3 * s + a) * C, C)], sem_c[s]).wait()

    def pass1(s):
        def grp(g, carry2):
            o = g * L
            cx = coords_v[pl.ds(3 * s * C + o, L)]
            cy = coords_v[pl.ds((3 * s + 1) * C + o, L)]
            cz = coords_v[pl.ds((3 * s + 2) * C + o, L)]
            x0r, xa, _, wx0, wx1 = _axis(cx, W)
            _, y0, y1, wy0, wy1 = _axis(cy, H)
            _, z0, z1, wz0, wz1 = _axis(cz, D)
            wlo = jnp.where(x0r >= 0, wx0, wx1)
            whi = jnp.where(x0r >= 0, wx1, 0.0)
            xa = (xa & 127) + (xa >> 7) * HALF
            iz0 = z0 * (H * 128)
            iz1 = z1 * (H * 128)
            iy0 = y0 * 128
            iy1 = y1 * 128
            corners = (
                (iz0 + iy0 + xa, wz0 * wy0),
                (iz0 + iy1 + xa, wz0 * wy1),
                (iz1 + iy0 + xa, wz1 * wy0),
                (iz1 + iy1 + xa, wz1 * wy1),
            )
            for k, (iv, wv) in enumerate(corners):
                idx_v[pl.ds((4 * s + k) * C + o, L)] = iv
                w_v[pl.ds((6 * s + k) * C + o, L)] = wv
            w_v[pl.ds((6 * s + 4) * C + o, L)] = wlo
            w_v[pl.ds((6 * s + 5) * C + o, L)] = whi
            return carry2

        lax.fori_loop(0, C // L, grp, 0, unroll=2)

    def fire_gather(s):
        pltpu.async_copy(grid_hbm.at[idx_v.at[pl.ds(4 * s * C, 4 * C)]],
                         vals_v.at[pl.ds(4 * s * C, 4 * C)], sem_g[s])

    def wait_gather(s):
        pltpu.make_async_copy(grid_hbm.at[pl.ds(0, 4 * C)],
                              vals_v.at[pl.ds(4 * s * C, 4 * C)],
                              sem_g[s]).wait()

    def pass2(ci, s):
        himask = jnp.uint32(0xFFFF0000)

        def grp2(g, carry2):
            o = g * L
            wlo = w_v[pl.ds((6 * s + 4) * C + o, L)]
            whi = w_v[pl.ds((6 * s + 5) * C + o, L)]
            acc = None
            for k in range(4):
                u = vals_v[pl.ds((4 * s + k) * C + o, L)]
                v0 = plsc.bitcast(u << 16, jnp.float32)
                v1 = plsc.bitcast(u & himask, jnp.float32)
                wzy = w_v[pl.ds((6 * s + k) * C + o, L)]
                term = wzy * (v0 * wlo + v1 * whi)
                acc = term if acc is None else acc + term
            out_v[pl.ds(s * C + o, L)] = acc
            return carry2

        lax.fori_loop(0, C // L, grp2, 0, unroll=2)
        pltpu.async_copy(out_v.at[pl.ds(s * C, C)],
                         out_hbm.at[pl.ds(base_pt + ci * C, C)], sem_o[s])

    def wait_out(s):
        pltpu.make_async_copy(out_v.at[pl.ds(s * C, C)],
                              out_hbm.at[pl.ds(base_pt, C)], sem_o[s]).wait()

    fire_coords(0, 0)

    def chunk_pair(i, carry):
        for s in (0, 1):
            t = 1 - s
            ci = 2 * i + s
            wait_coords(s)
            pass1(s)
            fire_gather(s)

            @pl.when(ci < NCHUNK - 1)
            def _():
                fire_coords(ci + 1, t)

            @pl.when(ci >= 3)
            def _():
                wait_out(t)

            @pl.when(ci >= 1)
            def _():
                wait_gather(t)
                pass2(ci - 1, t)

        return carry

    lax.fori_loop(0, NCHUNK // 2, chunk_pair, 0)
    s_last = (NCHUNK - 1) & 1
    wait_gather(s_last)
    pass2(NCHUNK - 1, s_last)
    wait_out(1 - s_last)
    wait_out(s_last)


R_BLK = 2048
N_RB = D * H // R_BLK
HALF = D * H * 128


def _pack_body(x_ref, b_ref, o_ref):
    x = x_ref[...]
    nxt = jnp.concatenate([x[:, 1:], b_ref[:, :1]], axis=1)
    lo = jax.lax.bitcast_convert_type(x.astype(jnp.bfloat16),
                                      jnp.uint16).astype(jnp.uint32)
    hi = jax.lax.bitcast_convert_type(nxt.astype(jnp.bfloat16),
                                      jnp.uint16).astype(jnp.uint32)
    o_ref[...] = lo | (hi << 16)


def _pack_grid(grid2):
    out = pl.pallas_call(
        _pack_body,
        grid=(N_RB, 2),
        in_specs=[pl.BlockSpec((R_BLK, 128), lambda i, j: (i, j)),
                  pl.BlockSpec((R_BLK, 128), lambda i, j: (i, 1))],
        out_specs=pl.BlockSpec((R_BLK, 128), lambda i, j: (j * N_RB + i, 0)),
        out_shape=jax.ShapeDtypeStruct((2 * D * H, 128), jnp.uint32),
    )(grid2, grid2)
    return out.reshape(D * H * W)


@jax.jit
def kernel(occupancy_grid, ray_bin_centers):
    grid_flat = _pack_grid(occupancy_grid.reshape(D * H, W))
    coords_flat = jnp.moveaxis(ray_bin_centers, -1, 0).reshape(3 * N)
    mesh = plsc.VectorSubcoreMesh(core_axis_name="c", subcore_axis_name="s")
    run = pl.kernel(
        _body,
        out_type=jax.ShapeDtypeStruct((N,), jnp.float32),
        mesh=mesh,
        compiler_params=pltpu.CompilerParams(needs_layout_passes=False),
        scratch_types=[
            pltpu.VMEM((2 * 3 * C,), jnp.float32),
            pltpu.VMEM((2 * 4 * C,), jnp.int32),
            pltpu.VMEM((2 * 6 * C,), jnp.float32),
            pltpu.VMEM((2 * 4 * C,), jnp.uint32),
            pltpu.VMEM((2 * C,), jnp.float32),
            pltpu.SemaphoreType.DMA,
            pltpu.SemaphoreType.DMA,
            pltpu.SemaphoreType.DMA,
            pltpu.SemaphoreType.DMA,
            pltpu.SemaphoreType.DMA,
            pltpu.SemaphoreType.DMA,
        ],
    )
    out = run(grid_flat, coords_flat)
    return out.reshape(N_RAYS, N_BINS)
